# baseline (speedup 1.0000x reference)
import jax
import jax.numpy as jnp
from jax import lax
from jax.experimental import pallas as pl
from jax.experimental.pallas import tpu as pltpu

_N = 96000
_D = 512
_PACK = 8
_KS = _PACK * _D
_NS = _N // _PACK
_LANES = _PACK * 5
_TILE = 800


def _tc_kernel(x_ref, w_ref, b_ref, out_ref):
    y = jnp.dot(x_ref[...], w_ref[...], preferred_element_type=jnp.float32)
    out_ref[...] = (y + b_ref[...]).reshape(1, _TILE, _LANES)


@jax.jit
def _run(x, wc, wp, bc, bp):
    eye = jnp.eye(_PACK, dtype=jnp.float32)
    coords_part = jnp.einsum('ij,do->idjo', eye, wc.T).reshape(_PACK, _D, 4 * _PACK)
    pres_part = jnp.einsum('ij,d->idj', eye, wp[0]).reshape(_PACK, _D, _PACK)
    w_s = jnp.concatenate([coords_part, pres_part], axis=2).reshape(_KS, _LANES)
    b_s = jnp.concatenate([jnp.tile(bc, _PACK), jnp.tile(bp, _PACK)]).reshape(1, _LANES)

    xs = x.reshape(_NS, _KS)
    nsteps = _NS // _TILE
    out = pl.pallas_call(
        _tc_kernel,
        grid=(nsteps,),
        in_specs=[
            pl.BlockSpec((_TILE, _KS), lambda i: (i, 0)),
            pl.BlockSpec((_KS, _LANES), lambda i: (0, 0)),
            pl.BlockSpec((1, _LANES), lambda i: (0, 0)),
        ],
        out_specs=pl.BlockSpec((1, _TILE, _LANES), lambda i: (i, 0, 0)),
        out_shape=jax.ShapeDtypeStruct((nsteps, _TILE, _LANES), jnp.float32),
        compiler_params=pltpu.CompilerParams(
            dimension_semantics=("arbitrary",),
        ),
    )(xs, w_s, b_s)
    out = out.reshape(_NS, _LANES)
    coords = out[:, :4 * _PACK].reshape(_N, 4)
    pres = out[:, 4 * _PACK:].reshape(_N, 1)
    return coords, pres


def kernel(local_features, W_coords, b_coords, W_pres, b_pres):
    B, C, R, D = local_features.shape
    x = local_features.reshape(B * C * R, D)
    coords, pres = _run(x, W_coords, W_pres, b_coords, b_pres)
    return (
        coords.reshape(B, C, R, 4),
        pres.reshape(B, C, R, 1),
    )


# speedup vs baseline: 2.0209x; 2.0209x over previous
import jax
import jax.numpy as jnp
from jax import lax
from jax.experimental import pallas as pl
from jax.experimental.pallas import tpu as pltpu
from jax.experimental.pallas import tpu_sc as plsc

_N_ROWS = 8 * 30 * 400
_D = 512
_NW = 32
_ROWS_BLK = 96
_GRP = 8
_N_SC = 30720
_N_TC = _N_ROWS - _N_SC
_ROWS_W = _N_SC // _NW
_NBLK = _ROWS_W // _ROWS_BLK
_TC_TILE = 2720
_MERGE_BLK = 3840


def _tc_kernel(x_ref, wc_ref, wp_ref, bc_ref, bp_ref, coords_ref, pres_ref):
    x = x_ref[...]
    w = jnp.concatenate([wc_ref[...], wp_ref[...]], axis=0)
    y = lax.dot_general(
        x, w,
        dimension_numbers=(((1,), (1,)), ((), ())),
        preferred_element_type=jnp.float32,
    )
    coords_ref[...] = y[:, 0:4] + bc_ref[...]
    pres_ref[...] = y[:, 4:5] + bp_ref[...]


def _tc_run(x, wc, wp, bc, bp):
    nsteps = _N_TC // _TC_TILE
    return pl.pallas_call(
        _tc_kernel,
        grid=(nsteps,),
        in_specs=[
            pl.BlockSpec((_TC_TILE, _D), lambda i: (i, 0)),
            pl.BlockSpec(wc.shape, lambda i: (0, 0)),
            pl.BlockSpec(wp.shape, lambda i: (0, 0)),
            pl.BlockSpec(bc.shape, lambda i: (0, 0)),
            pl.BlockSpec(bp.shape, lambda i: (0, 0)),
        ],
        out_specs=[
            pl.BlockSpec((_TC_TILE, 4), lambda i: (i, 0)),
            pl.BlockSpec((_TC_TILE, 1), lambda i: (i, 0)),
        ],
        out_shape=[
            jax.ShapeDtypeStruct((_N_ROWS, 4), jnp.float32),
            jax.ShapeDtypeStruct((_N_ROWS, 1), jnp.float32),
        ],
        compiler_params=pltpu.CompilerParams(
            dimension_semantics=("arbitrary",),
        ),
    )(x, wc, wp, bc, bp)


_GATHER_DNUMS = lax.GatherDimensionNumbers(
    offset_dims=(), collapsed_slice_dims=(0,), start_index_map=(0,))


def _lane_shuffle(v, idx):
    return lax.gather(v, idx[:, None], _GATHER_DNUMS, (1,),
                      mode=lax.GatherScatterMode.PROMISE_IN_BOUNDS)


def _sc_body(x_hbm, w_hbm, b_hbm, out_hbm,
             wv, bv, xb0, xb1, ob0, ob1,
             sx0, sx1, so0, so1):
    c = lax.axis_index("c")
    s = lax.axis_index("s")
    wid = s * 2 + c
    base = wid * _ROWS_W

    pltpu.sync_copy(w_hbm, wv)
    pltpu.sync_copy(b_hbm, bv)

    lane = lax.iota(jnp.int32, 16)

    def in_copy(blk, xb, sem):
        return pltpu.make_async_copy(
            x_hbm.at[pl.ds(_N_TC + base + blk * _ROWS_BLK, _ROWS_BLK), :],
            xb, sem)

    in_copy(0, xb0, sx0).start()
    in_copy(1, xb1, sx1).start()

    def compute_block(blk, xb, ob, sx, so):
        in_copy(blk, xb, sx).wait()

        @pl.when(blk >= 2)
        def _():
            pltpu.make_async_copy(
                ob, out_hbm.at[pl.ds(base, _ROWS_BLK), :], so).wait()

        bvec = bv[...]

        def grp_body(g, carry):
            r0 = g * _GRP
            accs = [[jnp.zeros((16,), jnp.float32) for _ in range(5)]
                    for _ in range(_GRP)]
            for dc in range(_D // 16):
                sl = pl.ds(dc * 16, 16)
                wvecs = [wv[o, sl] for o in range(5)]
                for r in range(_GRP):
                    xv = xb[r0 + r, sl]
                    for o in range(5):
                        accs[r][o] = accs[r][o] + xv * wvecs[o]
            for r in range(_GRP):
                comb = jnp.zeros((16,), jnp.float32)
                for o in range(5):
                    acc = accs[r][o]
                    for sh in (8, 4, 2, 1):
                        acc = acc + _lane_shuffle(acc, lane ^ sh)
                    comb = jnp.where(lane == o, acc + bvec, comb)
                ob[r0 + r, pl.ds(0, 16)] = comb
            return carry

        lax.fori_loop(0, _ROWS_BLK // _GRP, grp_body, 0)

        out_row = base + blk * _ROWS_BLK
        pltpu.make_async_copy(
            ob, out_hbm.at[pl.ds(out_row, _ROWS_BLK), :], so).start()

        @pl.when(blk + 2 < _NBLK)
        def _():
            in_copy(blk + 2, xb, sx).start()

    def blk_body(blk, carry):
        @pl.when(blk % 2 == 0)
        def _():
            compute_block(blk, xb0, ob0, sx0, so0)

        @pl.when(blk % 2 == 1)
        def _():
            compute_block(blk, xb1, ob1, sx1, so1)

        return carry

    lax.fori_loop(0, _NBLK, blk_body, 0)

    pltpu.make_async_copy(
        ob0, out_hbm.at[pl.ds(base, _ROWS_BLK), :], so0).wait()
    pltpu.make_async_copy(
        ob1, out_hbm.at[pl.ds(base, _ROWS_BLK), :], so1).wait()


_sc_run = pl.kernel(
    _sc_body,
    out_type=[
        jax.ShapeDtypeStruct((_N_SC, 16), jnp.float32),
    ],
    mesh=plsc.VectorSubcoreMesh(core_axis_name="c", subcore_axis_name="s"),
    scratch_types=[
        pltpu.VMEM((5, _D), jnp.float32),
        pltpu.VMEM((16,), jnp.float32),
        pltpu.VMEM((_ROWS_BLK, _D), jnp.float32),
        pltpu.VMEM((_ROWS_BLK, _D), jnp.float32),
        pltpu.VMEM((_ROWS_BLK, 16), jnp.float32),
        pltpu.VMEM((_ROWS_BLK, 16), jnp.float32),
        pltpu.SemaphoreType.DMA,
        pltpu.SemaphoreType.DMA,
        pltpu.SemaphoreType.DMA,
        pltpu.SemaphoreType.DMA,
    ],
)


def _merge_kernel(coords_in, pres_in, sc_ref, coords_ref, pres_ref):
    del coords_in, pres_in
    y = sc_ref[0]
    coords_ref[...] = y[:, 0:4]
    pres_ref[...] = y[:, 4:5]


def _merge_run(coords_full, pres_full, sc16):
    nsteps = _N_SC // _MERGE_BLK
    off = _N_TC // _MERGE_BLK
    return pl.pallas_call(
        _merge_kernel,
        grid=(nsteps,),
        in_specs=[
            pl.BlockSpec(memory_space=pl.ANY),
            pl.BlockSpec(memory_space=pl.ANY),
            pl.BlockSpec((1, _MERGE_BLK, 16), lambda i: (i, 0, 0)),
        ],
        out_specs=[
            pl.BlockSpec((_MERGE_BLK, 4), lambda i: (i + off, 0)),
            pl.BlockSpec((_MERGE_BLK, 1), lambda i: (i + off, 0)),
        ],
        out_shape=[
            jax.ShapeDtypeStruct((_N_ROWS, 4), jnp.float32),
            jax.ShapeDtypeStruct((_N_ROWS, 1), jnp.float32),
        ],
        input_output_aliases={0: 0, 1: 1},
        compiler_params=pltpu.CompilerParams(
            dimension_semantics=("arbitrary",),
        ),
    )(coords_full, pres_full, sc16)


@jax.jit
def _run(x, wc, wp, bc, bp):
    w5 = jnp.concatenate([wc, wp], axis=0)
    b5 = jnp.concatenate([bc.reshape(4), bp.reshape(1)])
    b16 = jnp.pad(b5, (0, 11))
    (sc_out,) = _sc_run(x, w5, b16)
    tc_c, tc_p = _tc_run(x, wc, wp, bc, bp)
    sc3 = sc_out.reshape(_N_SC // _MERGE_BLK, _MERGE_BLK, 16)
    coords, pres = _merge_run(tc_c, tc_p, sc3)
    return coords, pres


def kernel(local_features, W_coords, b_coords, W_pres, b_pres):
    B, C, R, D = local_features.shape
    x = local_features.reshape(B * C * R, D)
    coords, pres = _run(x, W_coords, W_pres,
                        b_coords.reshape(1, 4), b_pres.reshape(1, 1))
    return (
        coords.reshape(B, C, R, 4),
        pres.reshape(B, C, R, 1),
    )


# speedup vs baseline: 2.0360x; 1.0074x over previous
import jax
import jax.numpy as jnp
from jax import lax
from jax.experimental import pallas as pl
from jax.experimental.pallas import tpu as pltpu
from jax.experimental.pallas import tpu_sc as plsc

_N_ROWS = 8 * 30 * 400
_D = 512
_NW = 32
_ROWS_BLK = 96
_GRP = 8
_N_SC = 30720
_N_TC = _N_ROWS - _N_SC
_ROWS_W = _N_SC // _NW
_NBLK = _ROWS_W // _ROWS_BLK
_TC_TILE = 2720
_MERGE_BLK = 3840


def _tc_kernel(x_ref, wc_ref, wp_ref, bc_ref, bp_ref, coords_ref, pres_ref):
    x = x_ref[...]
    w = jnp.concatenate([wc_ref[...], wp_ref[...]], axis=0)
    y = lax.dot_general(
        x, w,
        dimension_numbers=(((1,), (1,)), ((), ())),
        preferred_element_type=jnp.float32,
    )
    coords_ref[...] = y[:, 0:4] + bc_ref[...]
    pres_ref[...] = y[:, 4:5] + bp_ref[...]


def _tc_run(x, wc, wp, bc, bp):
    nsteps = _N_TC // _TC_TILE
    return pl.pallas_call(
        _tc_kernel,
        grid=(nsteps,),
        in_specs=[
            pl.BlockSpec((_TC_TILE, _D), lambda i: (i, 0)),
            pl.BlockSpec(wc.shape, lambda i: (0, 0)),
            pl.BlockSpec(wp.shape, lambda i: (0, 0)),
            pl.BlockSpec(bc.shape, lambda i: (0, 0)),
            pl.BlockSpec(bp.shape, lambda i: (0, 0)),
        ],
        out_specs=[
            pl.BlockSpec((_TC_TILE, 4), lambda i: (i, 0)),
            pl.BlockSpec((_TC_TILE, 1), lambda i: (i, 0)),
        ],
        out_shape=[
            jax.ShapeDtypeStruct((_N_ROWS, 4), jnp.float32),
            jax.ShapeDtypeStruct((_N_ROWS, 1), jnp.float32),
        ],
        compiler_params=pltpu.CompilerParams(
            dimension_semantics=("arbitrary",),
        ),
    )(x, wc, wp, bc, bp)


_GATHER_DNUMS = lax.GatherDimensionNumbers(
    offset_dims=(), collapsed_slice_dims=(0,), start_index_map=(0,))


def _lane_shuffle(v, idx):
    return lax.gather(v, idx[:, None], _GATHER_DNUMS, (1,),
                      mode=lax.GatherScatterMode.PROMISE_IN_BOUNDS)


def _sc_body(x_hbm, w_hbm, b_hbm, out_hbm,
             wv, bv, xb0, xb1, ob0, ob1,
             sx0, sx1, so0, so1):
    c = lax.axis_index("c")
    s = lax.axis_index("s")
    wid = s * 2 + c
    base = wid * _ROWS_W

    pltpu.sync_copy(w_hbm, wv)
    pltpu.sync_copy(b_hbm, bv)

    lane = lax.iota(jnp.int32, 16)

    chunk = base // _MERGE_BLK
    coff = base % _MERGE_BLK

    def in_copy(blk, xb, sem):
        return pltpu.make_async_copy(
            x_hbm.at[pl.ds(_N_TC + base + blk * _ROWS_BLK, _ROWS_BLK), :],
            xb, sem)

    in_copy(0, xb0, sx0).start()
    in_copy(1, xb1, sx1).start()

    def compute_block(blk, xb, ob, sx, so):
        in_copy(blk, xb, sx).wait()

        @pl.when(blk >= 2)
        def _():
            pltpu.make_async_copy(
                ob, out_hbm.at[chunk, pl.ds(coff, _ROWS_BLK), :], so).wait()

        bvec = bv[...]

        def grp_body(g, carry):
            r0 = g * _GRP
            accs = [[jnp.zeros((16,), jnp.float32) for _ in range(5)]
                    for _ in range(_GRP)]
            for dc in range(_D // 16):
                sl = pl.ds(dc * 16, 16)
                wvecs = [wv[o, sl] for o in range(5)]
                for r in range(_GRP):
                    xv = xb[r0 + r, sl]
                    for o in range(5):
                        accs[r][o] = accs[r][o] + xv * wvecs[o]
            for r in range(_GRP):
                comb = jnp.zeros((16,), jnp.float32)
                for o in range(5):
                    acc = accs[r][o]
                    for sh in (8, 4, 2, 1):
                        acc = acc + _lane_shuffle(acc, lane ^ sh)
                    comb = jnp.where(lane == o, acc + bvec, comb)
                ob[r0 + r, pl.ds(0, 16)] = comb
            return carry

        lax.fori_loop(0, _ROWS_BLK // _GRP, grp_body, 0)

        pltpu.make_async_copy(
            ob, out_hbm.at[chunk, pl.ds(coff + blk * _ROWS_BLK, _ROWS_BLK), :],
            so).start()

        @pl.when(blk + 2 < _NBLK)
        def _():
            in_copy(blk + 2, xb, sx).start()

    def blk_body(blk, carry):
        @pl.when(blk % 2 == 0)
        def _():
            compute_block(blk, xb0, ob0, sx0, so0)

        @pl.when(blk % 2 == 1)
        def _():
            compute_block(blk, xb1, ob1, sx1, so1)

        return carry

    lax.fori_loop(0, _NBLK, blk_body, 0)

    pltpu.make_async_copy(
        ob0, out_hbm.at[chunk, pl.ds(coff, _ROWS_BLK), :], so0).wait()
    pltpu.make_async_copy(
        ob1, out_hbm.at[chunk, pl.ds(coff, _ROWS_BLK), :], so1).wait()


_sc_run = pl.kernel(
    _sc_body,
    out_type=[
        jax.ShapeDtypeStruct((_N_SC // _MERGE_BLK, _MERGE_BLK, 16),
                             jnp.float32),
    ],
    mesh=plsc.VectorSubcoreMesh(core_axis_name="c", subcore_axis_name="s"),
    scratch_types=[
        pltpu.VMEM((5, _D), jnp.float32),
        pltpu.VMEM((16,), jnp.float32),
        pltpu.VMEM((_ROWS_BLK, _D), jnp.float32),
        pltpu.VMEM((_ROWS_BLK, _D), jnp.float32),
        pltpu.VMEM((_ROWS_BLK, 16), jnp.float32),
        pltpu.VMEM((_ROWS_BLK, 16), jnp.float32),
        pltpu.SemaphoreType.DMA,
        pltpu.SemaphoreType.DMA,
        pltpu.SemaphoreType.DMA,
        pltpu.SemaphoreType.DMA,
    ],
)


def _merge_kernel(coords_in, pres_in, sc_ref, coords_ref, pres_ref):
    del coords_in, pres_in
    y = sc_ref[0]
    coords_ref[...] = y[:, 0:4]
    pres_ref[...] = y[:, 4:5]


def _merge_run(coords_full, pres_full, sc16):
    nsteps = _N_SC // _MERGE_BLK
    off = _N_TC // _MERGE_BLK
    return pl.pallas_call(
        _merge_kernel,
        grid=(nsteps,),
        in_specs=[
            pl.BlockSpec(memory_space=pl.ANY),
            pl.BlockSpec(memory_space=pl.ANY),
            pl.BlockSpec((1, _MERGE_BLK, 16), lambda i: (i, 0, 0)),
        ],
        out_specs=[
            pl.BlockSpec((_MERGE_BLK, 4), lambda i: (i + off, 0)),
            pl.BlockSpec((_MERGE_BLK, 1), lambda i: (i + off, 0)),
        ],
        out_shape=[
            jax.ShapeDtypeStruct((_N_ROWS, 4), jnp.float32),
            jax.ShapeDtypeStruct((_N_ROWS, 1), jnp.float32),
        ],
        input_output_aliases={0: 0, 1: 1},
        compiler_params=pltpu.CompilerParams(
            dimension_semantics=("arbitrary",),
        ),
    )(coords_full, pres_full, sc16)


@jax.jit
def _run(x, wc, wp, bc, bp):
    w5 = jnp.concatenate([wc, wp], axis=0)
    b5 = jnp.concatenate([bc.reshape(4), bp.reshape(1)])
    b16 = jnp.pad(b5, (0, 11))
    (sc3,) = _sc_run(x, w5, b16)
    tc_c, tc_p = _tc_run(x, wc, wp, bc, bp)
    coords, pres = _merge_run(tc_c, tc_p, sc3)
    return coords, pres


def kernel(local_features, W_coords, b_coords, W_pres, b_pres):
    B, C, R, D = local_features.shape
    x = local_features.reshape(B * C * R, D)
    coords, pres = _run(x, W_coords, W_pres,
                        b_coords.reshape(1, 4), b_pres.reshape(1, 1))
    return (
        coords.reshape(B, C, R, 4),
        pres.reshape(B, C, R, 1),
    )


# speedup vs baseline: 2.4935x; 1.2248x over previous
import jax
import jax.numpy as jnp
from jax import lax
from jax.experimental import pallas as pl
from jax.experimental.pallas import tpu as pltpu

_SLABS = 240
_R = 400
_D = 512
_SLAB_BLK = 10


def _tc_kernel(x_ref, wc_ref, wp_ref, bc_ref, bp_ref, coords_ref, pres_ref):
    x = x_ref[...].reshape(_SLAB_BLK * _R, _D)
    w = jnp.concatenate([wc_ref[...], wp_ref[...]], axis=0)
    y = lax.dot_general(
        x, w,
        dimension_numbers=(((1,), (1,)), ((), ())),
        preferred_element_type=jnp.float32,
    )
    coords_ref[...] = (y[:, 0:4] + bc_ref[...]).reshape(_SLAB_BLK, _R, 4)
    pres_ref[...] = (y[:, 4:5] + bp_ref[...]).reshape(_SLAB_BLK, _R, 1)


@jax.jit
def _run(x, wc, wp, bc, bp):
    nsteps = _SLABS // _SLAB_BLK
    coords, pres = pl.pallas_call(
        _tc_kernel,
        grid=(nsteps,),
        in_specs=[
            pl.BlockSpec((_SLAB_BLK, _R, _D), lambda i: (i, 0, 0)),
            pl.BlockSpec(wc.shape, lambda i: (0, 0)),
            pl.BlockSpec(wp.shape, lambda i: (0, 0)),
            pl.BlockSpec(bc.shape, lambda i: (0, 0)),
            pl.BlockSpec(bp.shape, lambda i: (0, 0)),
        ],
        out_specs=[
            pl.BlockSpec((_SLAB_BLK, _R, 4), lambda i: (i, 0, 0)),
            pl.BlockSpec((_SLAB_BLK, _R, 1), lambda i: (i, 0, 0)),
        ],
        out_shape=[
            jax.ShapeDtypeStruct((_SLABS, _R, 4), jnp.float32),
            jax.ShapeDtypeStruct((_SLABS, _R, 1), jnp.float32),
        ],
        compiler_params=pltpu.CompilerParams(
            dimension_semantics=("arbitrary",),
        ),
    )(x, wc, wp, bc, bp)
    return coords, pres


def kernel(local_features, W_coords, b_coords, W_pres, b_pres):
    B, C, R, D = local_features.shape
    x = local_features.reshape(B * C, R, D)
    coords, pres = _run(x, W_coords, W_pres,
                        b_coords.reshape(1, 4), b_pres.reshape(1, 1))
    return (
        coords.reshape(B, C, R, 4),
        pres.reshape(B, C, R, 1),
    )
